# Initial kernel scaffold; baseline (speedup 1.0000x reference)
import jax
import jax.numpy as jnp
from jax.experimental import pallas as pl


def kernel(input_ids, table):
    raise NotImplementedError("write your pallas kernel here")


# speedup vs baseline: 1.6834x; 1.6834x over previous
import functools

import jax
import jax.numpy as jnp
from jax import lax
from jax.experimental import pallas as pl
from jax.experimental.pallas import tpu as pltpu
from jax.experimental.pallas import tpu_sc as plsc


def _gather_kernel(n_rows, dim, nw, nchunk, chunk):
    mesh = plsc.VectorSubcoreMesh(core_axis_name="c", subcore_axis_name="s")
    nc = mesh.num_cores

    @functools.partial(
        pl.kernel,
        out_type=jax.ShapeDtypeStruct((n_rows, dim), jnp.float32),
        mesh=mesh,
        scratch_types=[
            pltpu.VMEM((nchunk, chunk), jnp.int32),
            pltpu.VMEM((chunk, dim), jnp.float32),
            pltpu.SemaphoreType.DMA,
        ],
        compiler_params=pltpu.CompilerParams(use_tc_tiling_on_sc=False),
    )
    def k(idx_hbm, table_hbm, out_hbm, idx_v, rows_v, gsem):
        wid = lax.axis_index("s") * nc + lax.axis_index("c")
        base = wid * (nchunk * chunk)
        pltpu.sync_copy(idx_hbm.at[wid], idx_v)

        @pl.loop(0, nchunk)
        def body(j):
            pltpu.async_copy(table_hbm.at[idx_v.at[j]], rows_v, gsem).wait()
            pltpu.sync_copy(rows_v, out_hbm.at[pl.ds(base + j * chunk, chunk)])

    return k


def kernel(input_ids, table):
    b, h = input_ids.shape
    v, d = table.shape
    n = b * h
    idx = input_ids.reshape(n).astype(jnp.int32)

    nw = 32
    chunk = 128
    rows_per_w = n // nw
    nchunk = rows_per_w // chunk
    assert rows_per_w * nw == n and nchunk * chunk == rows_per_w

    idx3 = idx.reshape(nw, nchunk, chunk)
    out = _gather_kernel(n, d, nw, nchunk, chunk)(idx3, table)
    return out.reshape(b, h, d)


# speedup vs baseline: 1.8765x; 1.1147x over previous
import functools

import jax
import jax.numpy as jnp
from jax import lax
from jax.experimental import pallas as pl
from jax.experimental.pallas import tpu as pltpu
from jax.experimental.pallas import tpu_sc as plsc


def _gather_kernel(n_rows, dim, nw, nchunk, chunk, nbuf):
    mesh = plsc.VectorSubcoreMesh(core_axis_name="c", subcore_axis_name="s")
    nc = mesh.num_cores
    ngroups = nchunk // nbuf
    assert ngroups * nbuf == nchunk

    @functools.partial(
        pl.kernel,
        out_type=jax.ShapeDtypeStruct((n_rows, dim), jnp.float32),
        mesh=mesh,
        scratch_types=[
            pltpu.VMEM((nchunk, chunk), jnp.int32),
            pltpu.VMEM((nbuf, chunk, dim), jnp.float32),
            pltpu.SemaphoreType.DMA,
            pltpu.SemaphoreType.DMA,
        ],
        compiler_params=pltpu.CompilerParams(use_tc_tiling_on_sc=False),
    )
    def k(idx_hbm, table_hbm, out_hbm, idx_v, rows_v, gsem, wsem):
        wid = lax.axis_index("s") * nc + lax.axis_index("c")
        base = wid * (nchunk * chunk)
        pltpu.sync_copy(idx_hbm.at[wid], idx_v)

        def start_gather(j, b):
            pltpu.async_copy(table_hbm.at[idx_v.at[j]], rows_v.at[b], gsem)

        def wait_gather(b):
            pltpu.make_async_copy(
                out_hbm.at[pl.ds(base, chunk)], rows_v.at[b], gsem
            ).wait()

        def start_wb(j, b):
            pltpu.async_copy(
                rows_v.at[b], out_hbm.at[pl.ds(base + j * chunk, chunk)], wsem
            )

        def wait_wb(b):
            pltpu.make_async_copy(
                rows_v.at[b], out_hbm.at[pl.ds(base, chunk)], wsem
            ).wait()

        for b in range(nbuf):
            start_gather(b, b)
        for b in range(nbuf):
            wait_gather(b)
            start_wb(b, b)

        @pl.loop(1, ngroups)
        def grp(g):
            j0 = g * nbuf
            for b in range(nbuf):
                wait_wb(b)
                start_gather(j0 + b, b)
            for b in range(nbuf):
                wait_gather(b)
                start_wb(j0 + b, b)

        for b in range(nbuf):
            wait_wb(b)

    return k


def kernel(input_ids, table):
    b, h = input_ids.shape
    v, d = table.shape
    n = b * h
    idx = input_ids.reshape(n).astype(jnp.int32)

    nw = 32
    chunk = 128
    rows_per_w = n // nw
    nchunk = rows_per_w // chunk
    assert rows_per_w * nw == n and nchunk * chunk == rows_per_w

    idx3 = idx.reshape(nw, nchunk, chunk)
    out = _gather_kernel(n, d, nw, nchunk, chunk, nbuf=8)(idx3, table)
    return out.reshape(b, h, d)
